# baseline (speedup 1.0000x reference)
import functools

import jax
import jax.numpy as jnp
from jax import lax
from jax.experimental import pallas as pl
from jax.experimental.pallas import tpu as pltpu
from jax.experimental.pallas import tpu_sc as plsc

N_NODES = 10000
N_EDGES = 320000
D = 128

NW = 32
EPW = N_EDGES // NW
C = 80
NCHUNK = EPW // C
G = C // 16


def _body(x_hbm, src_hbm, dst_hbm, out_hbm, idx_s, idx_d, u, v, o,
          sem_g, sem_i, sem_o):
    wid = lax.axis_index("s") * 2 + lax.axis_index("c")
    lanes = lax.iota(jnp.int32, 16)
    w0 = wid * EPW

    def issue_gather(b):
        pltpu.async_copy(x_hbm.at[idx_s.at[b]], u.at[b], sem_g)
        pltpu.async_copy(x_hbm.at[idx_d.at[b]], v.at[b], sem_g)

    def wait_gather():
        pltpu.make_async_copy(x_hbm.at[pl.ds(0, C)], u.at[0], sem_g).wait()
        pltpu.make_async_copy(x_hbm.at[pl.ds(0, C)], v.at[0], sem_g).wait()

    def issue_idx(c, b):
        base = w0 + c * C
        pltpu.async_copy(src_hbm.at[pl.ds(base, C)], idx_s.at[b], sem_i)
        pltpu.async_copy(dst_hbm.at[pl.ds(base, C)], idx_d.at[b], sem_i)

    def wait_idx():
        pltpu.make_async_copy(src_hbm.at[pl.ds(0, C)], idx_s.at[0], sem_i).wait()
        pltpu.make_async_copy(dst_hbm.at[pl.ds(0, C)], idx_d.at[0], sem_i).wait()

    def wait_out():
        pltpu.make_async_copy(out_hbm.at[pl.ds(0, C)], o.at[0], sem_o).wait()

    pltpu.sync_copy(src_hbm.at[pl.ds(w0, C)], idx_s.at[0])
    pltpu.sync_copy(dst_hbm.at[pl.ds(w0, C)], idx_d.at[0])
    issue_gather(0)
    issue_idx(1, 1)

    def chunk_body(c, _):
        b = lax.rem(c, 2)
        nb = 1 - b
        wait_gather()

        @pl.when(c + 1 < NCHUNK)
        def _():
            wait_idx()
            issue_gather(nb)

        @pl.when(c + 2 < NCHUNK)
        def _():
            issue_idx(c + 2, b)

        @pl.when(c >= 2)
        def _():
            wait_out()

        ub, vb, ob = u.at[b], v.at[b], o.at[b]

        def group_body(g, _):
            ev = g * 16 + lanes

            def blk_body(blk, res):
                wbase = blk * 8
                pr = []
                for k in range(8):
                    w = wbase + jnp.full((16,), k, jnp.int32)
                    gu = plsc.load_gather(ub, [ev, w])
                    gv = plsc.load_gather(vb, [ev, w])
                    pr.append(plsc.bitcast(gu, jnp.bfloat16)
                              * plsc.bitcast(gv, jnp.bfloat16))
                s = ((pr[0] + pr[1]) + (pr[2] + pr[3])) \
                    + ((pr[4] + pr[5]) + (pr[6] + pr[7]))
                a0, a1 = plsc.unpack(s, format=plsc.PackFormat.INTERLEAVED)
                return res + (a0 + a1)

            res = lax.fori_loop(0, 8, blk_body, jnp.zeros((16,), jnp.float32))
            ob[pl.ds(g * 16, 16)] = res
            return 0

        lax.fori_loop(0, G, group_body, 0)
        pltpu.async_copy(ob, out_hbm.at[pl.ds(w0 + c * C, C)], sem_o)
        return 0

    lax.fori_loop(0, NCHUNK, chunk_body, 0)
    wait_out()
    wait_out()


@jax.jit
def _run(x, src, dst):
    mesh = plsc.VectorSubcoreMesh(core_axis_name="c", subcore_axis_name="s")
    k = functools.partial(
        pl.kernel,
        mesh=mesh,
        compiler_params=pltpu.CompilerParams(
            needs_layout_passes=False, use_tc_tiling_on_sc=False),
        out_type=jax.ShapeDtypeStruct((N_EDGES,), jnp.float32),
        scratch_types=[
            pltpu.VMEM((2, C), jnp.int32),
            pltpu.VMEM((2, C), jnp.int32),
            pltpu.VMEM((2, C, D // 2), jnp.int32),
            pltpu.VMEM((2, C, D // 2), jnp.int32),
            pltpu.VMEM((2, C), jnp.float32),
            pltpu.SemaphoreType.DMA,
            pltpu.SemaphoreType.DMA,
            pltpu.SemaphoreType.DMA,
        ],
    )(_body)
    return k(x, src, dst)


def kernel(x, edge_index):
    src = edge_index[0].astype(jnp.int32)
    dst = edge_index[1].astype(jnp.int32)
    xb = x.astype(jnp.bfloat16)
    xi = lax.bitcast_convert_type(xb.reshape(N_NODES, D // 2, 2), jnp.int32)
    out = _run(xi, src, dst)
    return out.reshape(N_EDGES, 1)


# speedup vs baseline: 3.0239x; 3.0239x over previous
import functools

import jax
import jax.numpy as jnp
from jax import lax
from jax.experimental import pallas as pl
from jax.experimental.pallas import tpu as pltpu
from jax.experimental.pallas import tpu_sc as plsc

N_NODES = 10000
N_EDGES = 320000
D = 128

NW = 32
EPW = N_EDGES // NW
C = 400
NCHUNK = EPW // C
G = C // 16


def _body(x_hbm, src_hbm, dst_hbm, out_hbm, idx_s, idx_d, u, v, o, p,
          sem_g, sem_i, sem_o):
    wid = lax.axis_index("s") * 2 + lax.axis_index("c")
    lanes = lax.iota(jnp.int32, 16)
    w0 = wid * EPW

    def issue_gather(b):
        pltpu.async_copy(x_hbm.at[idx_s.at[b]], u.at[b], sem_g)
        pltpu.async_copy(x_hbm.at[idx_d.at[b]], v.at[b], sem_g)

    def wait_gather():
        pltpu.make_async_copy(x_hbm.at[pl.ds(0, C)], u.at[0], sem_g).wait()
        pltpu.make_async_copy(x_hbm.at[pl.ds(0, C)], v.at[0], sem_g).wait()

    def issue_idx(c, b):
        base = w0 + c * C
        pltpu.async_copy(src_hbm.at[pl.ds(base, C)], idx_s.at[b], sem_i)
        pltpu.async_copy(dst_hbm.at[pl.ds(base, C)], idx_d.at[b], sem_i)

    def wait_idx():
        pltpu.make_async_copy(src_hbm.at[pl.ds(0, C)], idx_s.at[0], sem_i).wait()
        pltpu.make_async_copy(dst_hbm.at[pl.ds(0, C)], idx_d.at[0], sem_i).wait()

    def wait_out():
        pltpu.make_async_copy(out_hbm.at[pl.ds(0, C)], o.at[0], sem_o).wait()

    pltpu.sync_copy(src_hbm.at[pl.ds(w0, C)], idx_s.at[0])
    pltpu.sync_copy(dst_hbm.at[pl.ds(w0, C)], idx_d.at[0])
    issue_gather(0)
    issue_idx(1, 1)

    def chunk_body(c, _):
        b = lax.rem(c, 2)
        nb = 1 - b
        wait_gather()

        @pl.when(c + 1 < NCHUNK)
        def _():
            wait_idx()
            issue_gather(nb)

        @pl.when(c + 2 < NCHUNK)
        def _():
            issue_idx(c + 2, b)

        @pl.when(c >= 2)
        def _():
            wait_out()

        ub, vb, ob = u.at[b], v.at[b], o.at[b]

        def group_body(g, _):
            for e16 in range(16):
                e = g * 16 + e16
                pr = []
                for k in range(D // 32):
                    uk = plsc.bitcast(ub[e, pl.ds(k * 16, 16)], jnp.bfloat16)
                    vk = plsc.bitcast(vb[e, pl.ds(k * 16, 16)], jnp.bfloat16)
                    pr.append(uk * vk)
                s = (pr[0] + pr[1]) + (pr[2] + pr[3])
                a0, a1 = plsc.unpack(s, format=plsc.PackFormat.INTERLEAVED)
                p[pl.ds(e16 * 16, 16)] = a0 + a1
            res = jnp.zeros((16,), jnp.float32)
            for l in range(16):
                res = res + plsc.load_gather(p, [lanes * 16 + l])
            ob[pl.ds(g * 16, 16)] = res
            return 0

        lax.fori_loop(0, G, group_body, 0)
        pltpu.async_copy(ob, out_hbm.at[pl.ds(w0 + c * C, C)], sem_o)
        return 0

    lax.fori_loop(0, NCHUNK, chunk_body, 0)
    wait_out()
    wait_out()


@jax.jit
def _run(x, src, dst):
    mesh = plsc.VectorSubcoreMesh(core_axis_name="c", subcore_axis_name="s")
    k = functools.partial(
        pl.kernel,
        mesh=mesh,
        compiler_params=pltpu.CompilerParams(
            needs_layout_passes=False, use_tc_tiling_on_sc=False),
        out_type=jax.ShapeDtypeStruct((N_EDGES,), jnp.float32),
        scratch_types=[
            pltpu.VMEM((2, C), jnp.int32),
            pltpu.VMEM((2, C), jnp.int32),
            pltpu.VMEM((2, C, D // 2), jnp.int32),
            pltpu.VMEM((2, C, D // 2), jnp.int32),
            pltpu.VMEM((2, C), jnp.float32),
            pltpu.VMEM((256,), jnp.float32),
            pltpu.SemaphoreType.DMA,
            pltpu.SemaphoreType.DMA,
            pltpu.SemaphoreType.DMA,
        ],
    )(_body)
    return k(x, src, dst)


def kernel(x, edge_index):
    src = edge_index[0].astype(jnp.int32)
    dst = edge_index[1].astype(jnp.int32)
    xb = x.astype(jnp.bfloat16)
    xi = lax.bitcast_convert_type(xb.reshape(N_NODES, D // 2, 2), jnp.int32)
    out = _run(xi, src, dst)
    return out.reshape(N_EDGES, 1)


# speedup vs baseline: 3.3566x; 1.1100x over previous
import functools

import jax
import jax.numpy as jnp
from jax import lax
from jax.experimental import pallas as pl
from jax.experimental.pallas import tpu as pltpu
from jax.experimental.pallas import tpu_sc as plsc

N_NODES = 10000
N_EDGES = 320000
D = 128

NW = 32
EPW = N_EDGES // NW
C = 80
NCHUNK = EPW // C
G = C // 16


def _emit_chunk_compute(ub, vb, ps, ob, lanes):
    l16 = lanes * 16
    n_edges = G * 16
    states = [None] * n_edges

    def load_thunks(e):
        st = {"lu": [None] * 4, "lv": [None] * 4}
        states[e] = st
        ths = []
        for k in range(4):
            def lu(st=st, k=k, e=e):
                st["lu"][k] = ub[e, pl.ds(k * 16, 16)]
            def lv(st=st, k=k, e=e):
                st["lv"][k] = vb[e, pl.ds(k * 16, 16)]
            ths += [lu, lv]
        return ths

    def comp_thunks(e):
        st = states[e]
        g, e16 = divmod(e, 16)
        pg = ps[g]
        ths = []
        for k in range(4):
            def mk(st=st, k=k):
                st["m%d" % k] = (plsc.bitcast(st["lu"][k], jnp.bfloat16)
                                 * plsc.bitcast(st["lv"][k], jnp.bfloat16))
            ths.append(mk)

        def s01(st=st):
            st["s01"] = st["m0"] + st["m1"]

        def s23(st=st):
            st["s23"] = st["m2"] + st["m3"]

        def sf(st=st):
            st["s"] = st["s01"] + st["s23"]

        def up(st=st):
            a0, a1 = plsc.unpack(st["s"], format=plsc.PackFormat.INTERLEAVED)
            st["t"] = a0 + a1

        def stt(st=st, pg=pg, e16=e16):
            pg[pl.ds(e16 * 16, 16)] = st["t"]

        ths += [s01, s23, sf, up, stt]
        if e16 == 15:
            acc = {}

            def t0(acc=acc, pg=pg):
                acc["r"] = plsc.load_gather(pg, [l16])

            ths.append(t0)
            for l in range(1, 16):
                def tl(acc=acc, pg=pg, l=l):
                    acc["r"] = acc["r"] + plsc.load_gather(pg, [l16 + l])
                ths.append(tl)

            def tw(acc=acc, g=g):
                ob[pl.ds(g * 16, 16)] = acc["r"]

            ths.append(tw)
        return ths

    for e in range(n_edges):
        ls = load_thunks(e)
        cs = comp_thunks(e - 1) if e > 0 else []
        i = j = 0
        while i < len(ls) or j < len(cs):
            if i < len(ls):
                ls[i]()
                i += 1
            if j < len(cs):
                cs[j]()
                j += 1
    for th in comp_thunks(n_edges - 1):
        th()


def _body(x_hbm, src_hbm, dst_hbm, out_hbm, idx_s, idx_d, u, v, o,
          p0, p1, p2, p3, p4, sem_g, sem_i, sem_o):
    ps = (p0, p1, p2, p3, p4)
    wid = lax.axis_index("s") * 2 + lax.axis_index("c")
    lanes = lax.iota(jnp.int32, 16)
    w0 = wid * EPW

    def issue_gather(b):
        pltpu.async_copy(x_hbm.at[idx_s.at[b]], u.at[b], sem_g)
        pltpu.async_copy(x_hbm.at[idx_d.at[b]], v.at[b], sem_g)

    def wait_gather():
        pltpu.make_async_copy(x_hbm.at[pl.ds(0, C)], u.at[0], sem_g).wait()
        pltpu.make_async_copy(x_hbm.at[pl.ds(0, C)], v.at[0], sem_g).wait()

    def issue_idx(c, b):
        base = w0 + c * C
        pltpu.async_copy(src_hbm.at[pl.ds(base, C)], idx_s.at[b], sem_i)
        pltpu.async_copy(dst_hbm.at[pl.ds(base, C)], idx_d.at[b], sem_i)

    def wait_idx():
        pltpu.make_async_copy(src_hbm.at[pl.ds(0, C)], idx_s.at[0], sem_i).wait()
        pltpu.make_async_copy(dst_hbm.at[pl.ds(0, C)], idx_d.at[0], sem_i).wait()

    def wait_out():
        pltpu.make_async_copy(out_hbm.at[pl.ds(0, C)], o.at[0], sem_o).wait()

    pltpu.sync_copy(src_hbm.at[pl.ds(w0, C)], idx_s.at[0])
    pltpu.sync_copy(dst_hbm.at[pl.ds(w0, C)], idx_d.at[0])
    issue_gather(0)
    issue_idx(1, 1)

    def chunk_body(c, _):
        b = lax.rem(c, 2)
        nb = 1 - b
        wait_gather()

        @pl.when(c + 1 < NCHUNK)
        def _():
            wait_idx()
            issue_gather(nb)

        @pl.when(c + 2 < NCHUNK)
        def _():
            issue_idx(c + 2, b)

        @pl.when(c >= 2)
        def _():
            wait_out()

        ub, vb, ob = u.at[b], v.at[b], o.at[b]
        _emit_chunk_compute(ub, vb, ps, ob, lanes)
        pltpu.async_copy(ob, out_hbm.at[pl.ds(w0 + c * C, C)], sem_o)
        return 0

    lax.fori_loop(0, NCHUNK, chunk_body, 0)
    wait_out()
    wait_out()


@jax.jit
def _run(x, src, dst):
    mesh = plsc.VectorSubcoreMesh(core_axis_name="c", subcore_axis_name="s")
    k = functools.partial(
        pl.kernel,
        mesh=mesh,
        compiler_params=pltpu.CompilerParams(
            needs_layout_passes=False, use_tc_tiling_on_sc=False),
        out_type=jax.ShapeDtypeStruct((N_EDGES,), jnp.float32),
        scratch_types=[
            pltpu.VMEM((2, C), jnp.int32),
            pltpu.VMEM((2, C), jnp.int32),
            pltpu.VMEM((2, C, D // 2), jnp.int32),
            pltpu.VMEM((2, C, D // 2), jnp.int32),
            pltpu.VMEM((2, C), jnp.float32),
            pltpu.VMEM((256,), jnp.float32),
            pltpu.VMEM((256,), jnp.float32),
            pltpu.VMEM((256,), jnp.float32),
            pltpu.VMEM((256,), jnp.float32),
            pltpu.VMEM((256,), jnp.float32),
            pltpu.SemaphoreType.DMA,
            pltpu.SemaphoreType.DMA,
            pltpu.SemaphoreType.DMA,
        ],
    )(_body)
    return k(x, src, dst)


def kernel(x, edge_index):
    src = edge_index[0].astype(jnp.int32)
    dst = edge_index[1].astype(jnp.int32)
    xb = x.astype(jnp.bfloat16)
    xi = lax.bitcast_convert_type(xb.reshape(N_NODES, D // 2, 2), jnp.int32)
    out = _run(xi, src, dst)
    return out.reshape(N_EDGES, 1)


# speedup vs baseline: 3.3660x; 1.0028x over previous
import functools

import jax
import jax.numpy as jnp
from jax import lax
from jax.experimental import pallas as pl
from jax.experimental.pallas import tpu as pltpu
from jax.experimental.pallas import tpu_sc as plsc

N_NODES = 10000
N_EDGES = 320000
D = 128

NW = 32
EPW = N_EDGES // NW
C = 80
NCHUNK = EPW // C
G = C // 16


def _emit_chunk_compute(ub, vb, ob, lanes):
    n_edges = G * 16
    states = [None] * n_edges

    def load_thunks(e):
        st = {"lu": [None] * 4, "lv": [None] * 4}
        states[e] = st
        ths = []
        for k in range(4):
            def lu(st=st, k=k, e=e):
                st["lu"][k] = ub[e, pl.ds(k * 16, 16)]
            def lv(st=st, k=k, e=e):
                st["lv"][k] = vb[e, pl.ds(k * 16, 16)]
            ths += [lu, lv]
        return ths

    gaccs = [{"v": None} for _ in range(G)]

    def comp_thunks(e):
        st = states[e]
        g, e16 = divmod(e, 16)
        gacc = gaccs[g]
        ths = []
        for k in range(4):
            def mk(st=st, k=k):
                st["m%d" % k] = (plsc.bitcast(st["lu"][k], jnp.bfloat16)
                                 * plsc.bitcast(st["lv"][k], jnp.bfloat16))
            ths.append(mk)

        def s01(st=st):
            st["s01"] = st["m0"] + st["m1"]

        def s23(st=st):
            st["s23"] = st["m2"] + st["m3"]

        def sf(st=st):
            st["s"] = st["s01"] + st["s23"]

        def up(st=st):
            a0, a1 = plsc.unpack(st["s"], format=plsc.PackFormat.INTERLEAVED)
            st["t"] = a0 + a1

        def red(st=st):
            st["r"] = jnp.sum(st["t"])

        def mrg(st=st, gacc=gacc, e16=e16):
            if gacc["v"] is None:
                gacc["v"] = jnp.where(lanes == e16, st["r"],
                                      jnp.zeros((16,), jnp.float32))
            else:
                gacc["v"] = jnp.where(lanes == e16, st["r"], gacc["v"])

        ths += [s01, s23, sf, up, red, mrg]
        if e16 == 15:
            def tw(gacc=gacc, g=g):
                ob[pl.ds(g * 16, 16)] = gacc["v"]

            ths.append(tw)
        return ths

    for e in range(n_edges):
        ls = load_thunks(e)
        cs = comp_thunks(e - 1) if e > 0 else []
        i = j = 0
        while i < len(ls) or j < len(cs):
            if i < len(ls):
                ls[i]()
                i += 1
            if j < len(cs):
                cs[j]()
                j += 1
    for th in comp_thunks(n_edges - 1):
        th()


def _body(x_hbm, src_hbm, dst_hbm, out_hbm, idx_s, idx_d, u, v, o,
          sem_g, sem_i, sem_o):
    wid = lax.axis_index("s") * 2 + lax.axis_index("c")
    lanes = lax.iota(jnp.int32, 16)
    w0 = wid * EPW

    def issue_gather(b):
        pltpu.async_copy(x_hbm.at[idx_s.at[b]], u.at[b], sem_g)
        pltpu.async_copy(x_hbm.at[idx_d.at[b]], v.at[b], sem_g)

    def wait_gather():
        pltpu.make_async_copy(x_hbm.at[pl.ds(0, C)], u.at[0], sem_g).wait()
        pltpu.make_async_copy(x_hbm.at[pl.ds(0, C)], v.at[0], sem_g).wait()

    def issue_idx(c, b):
        base = w0 + c * C
        pltpu.async_copy(src_hbm.at[pl.ds(base, C)], idx_s.at[b], sem_i)
        pltpu.async_copy(dst_hbm.at[pl.ds(base, C)], idx_d.at[b], sem_i)

    def wait_idx():
        pltpu.make_async_copy(src_hbm.at[pl.ds(0, C)], idx_s.at[0], sem_i).wait()
        pltpu.make_async_copy(dst_hbm.at[pl.ds(0, C)], idx_d.at[0], sem_i).wait()

    def wait_out():
        pltpu.make_async_copy(out_hbm.at[pl.ds(0, C)], o.at[0], sem_o).wait()

    pltpu.sync_copy(src_hbm.at[pl.ds(w0, C)], idx_s.at[0])
    pltpu.sync_copy(dst_hbm.at[pl.ds(w0, C)], idx_d.at[0])
    issue_gather(0)
    issue_idx(1, 1)

    def chunk_body(c, _):
        b = lax.rem(c, 2)
        nb = 1 - b
        wait_gather()

        @pl.when(c + 1 < NCHUNK)
        def _():
            wait_idx()
            issue_gather(nb)

        @pl.when(c + 2 < NCHUNK)
        def _():
            issue_idx(c + 2, b)

        @pl.when(c >= 2)
        def _():
            wait_out()

        ub, vb, ob = u.at[b], v.at[b], o.at[b]
        _emit_chunk_compute(ub, vb, ob, lanes)
        pltpu.async_copy(ob, out_hbm.at[pl.ds(w0 + c * C, C)], sem_o)
        return 0

    lax.fori_loop(0, NCHUNK, chunk_body, 0)
    wait_out()
    wait_out()


@jax.jit
def _run(x, src, dst):
    mesh = plsc.VectorSubcoreMesh(core_axis_name="c", subcore_axis_name="s")
    k = functools.partial(
        pl.kernel,
        mesh=mesh,
        compiler_params=pltpu.CompilerParams(
            needs_layout_passes=False, use_tc_tiling_on_sc=False),
        out_type=jax.ShapeDtypeStruct((N_EDGES,), jnp.float32),
        scratch_types=[
            pltpu.VMEM((2, C), jnp.int32),
            pltpu.VMEM((2, C), jnp.int32),
            pltpu.VMEM((2, C, D // 2), jnp.int32),
            pltpu.VMEM((2, C, D // 2), jnp.int32),
            pltpu.VMEM((2, C), jnp.float32),
            pltpu.SemaphoreType.DMA,
            pltpu.SemaphoreType.DMA,
            pltpu.SemaphoreType.DMA,
        ],
    )(_body)
    return k(x, src, dst)


def kernel(x, edge_index):
    src = edge_index[0].astype(jnp.int32)
    dst = edge_index[1].astype(jnp.int32)
    xb = x.astype(jnp.bfloat16)
    xi = lax.bitcast_convert_type(xb.reshape(N_NODES, D // 2, 2), jnp.int32)
    out = _run(xi, src, dst)
    return out.reshape(N_EDGES, 1)


# speedup vs baseline: 3.3699x; 1.0011x over previous
import functools

import jax
import jax.numpy as jnp
from jax import lax
from jax.experimental import pallas as pl
from jax.experimental.pallas import tpu as pltpu
from jax.experimental.pallas import tpu_sc as plsc

N_NODES = 10000
N_EDGES = 320000
D = 128

NW = 32
EPW = N_EDGES // NW
C = 80
NCHUNK = EPW // C
G = C // 16


def _emit_chunk_compute(ub, vb, ob, lanes):
    n_edges = G * 16
    states = [None] * n_edges

    def load_thunks(e):
        st = {"lu": [None] * 4, "lv": [None] * 4}
        states[e] = st
        ths = []
        for k in range(4):
            def lu(st=st, k=k, e=e):
                st["lu"][k] = ub[e, pl.ds(k * 16, 16)]
            def lv(st=st, k=k, e=e):
                st["lv"][k] = vb[e, pl.ds(k * 16, 16)]
            ths += [lu, lv]
        return ths

    gaccs = [{"v": None} for _ in range(G)]

    def comp_thunks(e):
        st = states[e]
        g, e16 = divmod(e, 16)
        gacc = gaccs[g]
        ths = []
        for k in range(4):
            def mk(st=st, k=k):
                st["m%d" % k] = (plsc.bitcast(st["lu"][k], jnp.bfloat16)
                                 * plsc.bitcast(st["lv"][k], jnp.bfloat16))
            ths.append(mk)

        def s01(st=st):
            st["s01"] = st["m0"] + st["m1"]

        def s23(st=st):
            st["s23"] = st["m2"] + st["m3"]

        def sf(st=st):
            st["s"] = st["s01"] + st["s23"]

        def up(st=st):
            a0, a1 = plsc.unpack(st["s"], format=plsc.PackFormat.INTERLEAVED)
            st["t"] = a0 + a1

        def red(st=st):
            st["r"] = jnp.sum(st["t"])

        def mrg(st=st, gacc=gacc, e16=e16):
            if gacc["v"] is None:
                gacc["v"] = jnp.where(lanes == e16, st["r"],
                                      jnp.zeros((16,), jnp.float32))
            else:
                gacc["v"] = jnp.where(lanes == e16, st["r"], gacc["v"])

        ths += [s01, s23, sf, up, red, mrg]
        if e16 == 15:
            def tw(gacc=gacc, g=g):
                ob[pl.ds(g * 16, 16)] = gacc["v"]

            ths.append(tw)
        return ths

    for e in range(n_edges):
        ls = load_thunks(e)
        cs = comp_thunks(e - 1) if e > 0 else []
        i = j = 0
        while i < len(ls) or j < len(cs):
            if i < len(ls):
                ls[i]()
                i += 1
            if j < len(cs):
                cs[j]()
                j += 1
    for th in comp_thunks(n_edges - 1):
        th()


def _body(x_hbm, src_hbm, dst_hbm, out_hbm, idx_s, idx_d, u, v, o,
          sem_g, sem_i, sem_o):
    wid = lax.axis_index("s") * 2 + lax.axis_index("c")
    lanes = lax.iota(jnp.int32, 16)
    w0 = wid * EPW

    H = C // 2

    def issue_gather(b):
        for h in range(2):
            sl = pl.ds(h * H, H)
            pltpu.async_copy(x_hbm.at[idx_s.at[b].at[sl]], u.at[b].at[sl], sem_g)
            pltpu.async_copy(x_hbm.at[idx_d.at[b].at[sl]], v.at[b].at[sl], sem_g)

    def wait_gather():
        for h in range(2):
            sl = pl.ds(0, H)
            pltpu.make_async_copy(x_hbm.at[sl], u.at[0].at[sl], sem_g).wait()
            pltpu.make_async_copy(x_hbm.at[sl], v.at[0].at[sl], sem_g).wait()

    def issue_idx(c, b):
        base = w0 + c * C
        pltpu.async_copy(src_hbm.at[pl.ds(base, C)], idx_s.at[b], sem_i)
        pltpu.async_copy(dst_hbm.at[pl.ds(base, C)], idx_d.at[b], sem_i)

    def wait_idx():
        pltpu.make_async_copy(src_hbm.at[pl.ds(0, C)], idx_s.at[0], sem_i).wait()
        pltpu.make_async_copy(dst_hbm.at[pl.ds(0, C)], idx_d.at[0], sem_i).wait()

    def wait_out():
        pltpu.make_async_copy(out_hbm.at[pl.ds(0, C)], o.at[0], sem_o).wait()

    pltpu.sync_copy(src_hbm.at[pl.ds(w0, C)], idx_s.at[0])
    pltpu.sync_copy(dst_hbm.at[pl.ds(w0, C)], idx_d.at[0])
    issue_gather(0)
    issue_idx(1, 1)

    def chunk_body(c, _):
        b = lax.rem(c, 2)
        nb = 1 - b
        wait_gather()

        @pl.when(c + 1 < NCHUNK)
        def _():
            wait_idx()
            issue_gather(nb)

        @pl.when(c + 2 < NCHUNK)
        def _():
            issue_idx(c + 2, b)

        @pl.when(c >= 2)
        def _():
            wait_out()

        ub, vb, ob = u.at[b], v.at[b], o.at[b]
        _emit_chunk_compute(ub, vb, ob, lanes)
        pltpu.async_copy(ob, out_hbm.at[pl.ds(w0 + c * C, C)], sem_o)
        return 0

    lax.fori_loop(0, NCHUNK, chunk_body, 0)
    wait_out()
    wait_out()


@jax.jit
def _run(x, src, dst):
    mesh = plsc.VectorSubcoreMesh(core_axis_name="c", subcore_axis_name="s")
    k = functools.partial(
        pl.kernel,
        mesh=mesh,
        compiler_params=pltpu.CompilerParams(
            needs_layout_passes=False, use_tc_tiling_on_sc=False),
        out_type=jax.ShapeDtypeStruct((N_EDGES,), jnp.float32),
        scratch_types=[
            pltpu.VMEM((2, C), jnp.int32),
            pltpu.VMEM((2, C), jnp.int32),
            pltpu.VMEM((2, C, D // 2), jnp.int32),
            pltpu.VMEM((2, C, D // 2), jnp.int32),
            pltpu.VMEM((2, C), jnp.float32),
            pltpu.SemaphoreType.DMA,
            pltpu.SemaphoreType.DMA,
            pltpu.SemaphoreType.DMA,
        ],
    )(_body)
    return k(x, src, dst)


def kernel(x, edge_index):
    src = edge_index[0].astype(jnp.int32)
    dst = edge_index[1].astype(jnp.int32)
    xb = x.astype(jnp.bfloat16)
    xi = lax.bitcast_convert_type(xb.reshape(N_NODES, D // 2, 2), jnp.int32)
    out = _run(xi, src, dst)
    return out.reshape(N_EDGES, 1)
